# baseline (speedup 1.0000x reference)
import jax
import jax.numpy as jnp
from jax import lax
from jax.experimental import pallas as pl
from jax.experimental.pallas import tpu as pltpu
from jax.experimental.pallas import tpu_sc as plsc

_NUM_SLOTS = 8
_DM = 768
_DFF = 1536
_NC = 2
_NS = 16
_NW = _NC * _NS
_PARTS = _NW // _NUM_SLOTS

_ROWS13 = 2 * _DFF
_HEAD13 = 768
_TAIL13 = _ROWS13 - _HEAD13
_TROWS13 = _TAIL13 // _PARTS
_CH13 = 48
_N13 = _TROWS13 // _CH13


def _ring_copy(src, dst, base, ch, nch, b0, b1, si0, si1, so0, so1):
    def start_in(j, buf, sem):
        pltpu.async_copy(src.at[pl.ds(base + j * ch, ch), :], buf, sem)

    def start_out(j, buf, sem):
        pltpu.async_copy(buf, dst.at[pl.ds(base + j * ch, ch), :], sem)

    def win(buf, sem):
        pltpu.make_async_copy(src.at[pl.ds(base, ch), :], buf, sem).wait()

    def wout(buf, sem):
        pltpu.make_async_copy(buf, dst.at[pl.ds(base, ch), :], sem).wait()

    K = nch // 2
    start_in(0, b0, si0)

    def body(k, carry):
        j = 2 * k

        @pl.when(k > 0)
        def _():
            wout(b1, so1)
        start_in(j + 1, b1, si1)
        win(b0, si0)
        start_out(j, b0, so0)

        @pl.when(k < K - 1)
        def _():
            wout(b0, so0)
            start_in(j + 2, b0, si0)
        win(b1, si1)
        start_out(j + 1, b1, so1)
        return carry

    lax.fori_loop(0, K, body, 0)
    wout(b0, so0)
    wout(b1, so1)


def _sc_body(w13, b13, b2, eids, sids, o13, ob13, ob2,
             ids_v, a0, a1, bv,
             si0, si1, so0, so1):
    wid = lax.axis_index("s") * _NC + lax.axis_index("c")

    pltpu.sync_copy(eids, ids_v.at[pl.ds(0, 8)])
    pltpu.sync_copy(sids, ids_v.at[pl.ds(8, 8)])

    def pick(k):
        return ids_v[pl.ds(k, 16)][0]

    slot = wid // _PARTS
    part = wid % _PARTS
    eid = pick(slot)
    dst = pick(8 + slot)

    @pl.when(wid < _NUM_SLOTS)
    def _():
        beid = pick(wid)
        bdst = pick(8 + wid)
        pltpu.sync_copy(b13.at[beid], bv)
        pltpu.sync_copy(bv, ob13.at[bdst])
        pltpu.sync_copy(b2.at[beid], bv.at[pl.ds(0, _DM)])
        pltpu.sync_copy(bv.at[pl.ds(0, _DM)], ob2.at[bdst])

    _ring_copy(w13.at[eid], o13.at[dst], _HEAD13 + part * _TROWS13,
               _CH13, _N13, a0, a1, si0, si1, so0, so1)


_sc_fetch = pl.kernel(
    _sc_body,
    out_type=(
        jax.ShapeDtypeStruct((_NUM_SLOTS, _ROWS13, _DM), jnp.float32),
        jax.ShapeDtypeStruct((_NUM_SLOTS, _ROWS13), jnp.float32),
        jax.ShapeDtypeStruct((_NUM_SLOTS, _DM), jnp.float32),
    ),
    mesh=plsc.VectorSubcoreMesh(
        core_axis_name="c", subcore_axis_name="s",
        num_cores=_NC, num_subcores=_NS),
    scratch_types=[
        pltpu.VMEM((32,), jnp.int32),
        pltpu.VMEM((_CH13, _DM), jnp.float32),
        pltpu.VMEM((_CH13, _DM), jnp.float32),
        pltpu.VMEM((_ROWS13,), jnp.float32),
        pltpu.SemaphoreType.DMA,
        pltpu.SemaphoreType.DMA,
        pltpu.SemaphoreType.DMA,
        pltpu.SemaphoreType.DMA,
    ],
)


def _tc_body(eids_ref, sids_ref, w2_ref, o2_ref):
    o2_ref[...] = w2_ref[...]


def _tc_fetch(w2, eids, sids):
    return pl.pallas_call(
        _tc_body,
        grid_spec=pltpu.PrefetchScalarGridSpec(
            num_scalar_prefetch=2,
            grid=(_NUM_SLOTS,),
            in_specs=[
                pl.BlockSpec((1, _DM, _DFF),
                             lambda i, eids, sids: (eids[i], 0, 0)),
            ],
            out_specs=pl.BlockSpec((1, _DM, _DFF),
                                   lambda i, eids, sids: (sids[i], 0, 0)),
        ),
        out_shape=jax.ShapeDtypeStruct((_NUM_SLOTS, _DM, _DFF),
                                       jnp.float32),
    )(eids, sids, w2)


def _tc_head_body(eids_ref, sids_ref, o13_hbm, w13_ref, out_ref):
    del o13_hbm
    out_ref[...] = w13_ref[...]


def _tc_head_fetch(o13_partial, w13, eids, sids):
    return pl.pallas_call(
        _tc_head_body,
        grid_spec=pltpu.PrefetchScalarGridSpec(
            num_scalar_prefetch=2,
            grid=(_NUM_SLOTS,),
            in_specs=[
                pl.BlockSpec(memory_space=pltpu.MemorySpace.HBM),
                pl.BlockSpec((1, _HEAD13, _DM),
                             lambda i, eids, sids: (eids[i], 0, 0)),
            ],
            out_specs=pl.BlockSpec((1, _HEAD13, _DM),
                                   lambda i, eids, sids: (sids[i], 0, 0)),
        ),
        out_shape=jax.ShapeDtypeStruct((_NUM_SLOTS, _ROWS13, _DM),
                                       jnp.float32),
        input_output_aliases={2: 0},
    )(eids, sids, o13_partial, w13)


@jax.jit
def _fetch(w13, b13, w2, b2, eids, sids):
    o13_partial, ob13, ob2 = _sc_fetch(w13, b13, b2, eids, sids)
    o2 = _tc_fetch(w2, eids, sids)
    o13 = _tc_head_fetch(o13_partial, w13, eids, sids)
    return o13, ob13, o2, ob2


def kernel(w13_weight, w13_bias, w2_weight, w2_bias, expert_ids, slot_ids):
    expert_ids = expert_ids.reshape(-1).astype(jnp.int32)
    slot_ids = slot_ids.reshape(-1).astype(jnp.int32)
    o13, ob13, o2, ob2 = _fetch(w13_weight, w13_bias, w2_weight, w2_bias,
                                expert_ids, slot_ids)
    return (o13, ob13, o2, ob2)


# speedup vs baseline: 1.1051x; 1.1051x over previous
import jax
import jax.numpy as jnp
from jax import lax
from jax.experimental import pallas as pl
from jax.experimental.pallas import tpu as pltpu
from jax.experimental.pallas import tpu_sc as plsc

_NUM_SLOTS = 8
_DM = 768
_DFF = 1536
_NC = 2
_NS = 16
_NW = _NC * _NS
_PARTS = _NW // _NUM_SLOTS

_ROWS13 = 2 * _DFF
_TROWS2 = _DM // _PARTS
_CH2 = 16
_N2 = _TROWS2 // _CH2


def _ring_copy(src, dst, base, ch, nch, b0, b1, si0, si1, so0, so1):
    def start_in(j, buf, sem):
        pltpu.async_copy(src.at[pl.ds(base + j * ch, ch), :], buf, sem)

    def start_out(j, buf, sem):
        pltpu.async_copy(buf, dst.at[pl.ds(base + j * ch, ch), :], sem)

    def win(buf, sem):
        pltpu.make_async_copy(src.at[pl.ds(base, ch), :], buf, sem).wait()

    def wout(buf, sem):
        pltpu.make_async_copy(buf, dst.at[pl.ds(base, ch), :], sem).wait()

    K = nch // 2
    start_in(0, b0, si0)

    def body(k, carry):
        j = 2 * k

        @pl.when(k > 0)
        def _():
            wout(b1, so1)
        start_in(j + 1, b1, si1)
        win(b0, si0)
        start_out(j, b0, so0)

        @pl.when(k < K - 1)
        def _():
            wout(b0, so0)
            start_in(j + 2, b0, si0)
        win(b1, si1)
        start_out(j + 1, b1, so1)
        return carry

    lax.fori_loop(0, K, body, 0)
    wout(b0, so0)
    wout(b1, so1)


def _sc_body(w2, b13, b2, eids, sids,
             o2, ob13, ob2,
             ids_v, c0, c1, bv,
             si0, si1, so0, so1):
    wid = lax.axis_index("s") * _NC + lax.axis_index("c")

    pltpu.sync_copy(eids, ids_v.at[pl.ds(0, 8)])
    pltpu.sync_copy(sids, ids_v.at[pl.ds(8, 8)])

    def pick(k):
        return ids_v[pl.ds(k, 16)][0]

    slot = wid // _PARTS
    part = wid % _PARTS
    eid = pick(slot)
    dst = pick(8 + slot)

    @pl.when(wid < _NUM_SLOTS)
    def _():
        beid = pick(wid)
        bdst = pick(8 + wid)
        pltpu.sync_copy(b13.at[beid], bv)
        pltpu.sync_copy(bv, ob13.at[bdst])
        pltpu.sync_copy(b2.at[beid], bv.at[pl.ds(0, _DM)])
        pltpu.sync_copy(bv.at[pl.ds(0, _DM)], ob2.at[bdst])

    _ring_copy(w2.at[eid], o2.at[dst], part * _TROWS2, _CH2, _N2,
               c0, c1, si0, si1, so0, so1)


_sc_fetch = pl.kernel(
    _sc_body,
    out_type=(
        jax.ShapeDtypeStruct((_NUM_SLOTS, _DM, _DFF), jnp.float32),
        jax.ShapeDtypeStruct((_NUM_SLOTS, _ROWS13), jnp.float32),
        jax.ShapeDtypeStruct((_NUM_SLOTS, _DM), jnp.float32),
    ),
    mesh=plsc.VectorSubcoreMesh(
        core_axis_name="c", subcore_axis_name="s",
        num_cores=_NC, num_subcores=_NS),
    scratch_types=[
        pltpu.VMEM((32,), jnp.int32),
        pltpu.VMEM((_CH2, _DFF), jnp.float32),
        pltpu.VMEM((_CH2, _DFF), jnp.float32),
        pltpu.VMEM((_ROWS13,), jnp.float32),
        pltpu.SemaphoreType.DMA,
        pltpu.SemaphoreType.DMA,
        pltpu.SemaphoreType.DMA,
        pltpu.SemaphoreType.DMA,
    ],
)

_TC_NB = 6
_TC_SPLIT = 4
_TC_PR = _ROWS13 // _TC_SPLIT
_TC_NITEMS = _NUM_SLOTS * _TC_SPLIT


def _tc_body(eids_ref, sids_ref, in_hbm, out_hbm, *rest):
    bufs = rest[:_TC_NB]
    isems = rest[_TC_NB:2 * _TC_NB]
    osems = rest[2 * _TC_NB:3 * _TC_NB]

    def src(t):
        i, h = divmod(t, _TC_SPLIT)
        return in_hbm.at[eids_ref[i], pl.ds(h * _TC_PR, _TC_PR), :]

    def dstref(t):
        i, h = divmod(t, _TC_SPLIT)
        return out_hbm.at[sids_ref[i], pl.ds(h * _TC_PR, _TC_PR), :]

    for t in range(_TC_NITEMS + 2):
        if t < _TC_NITEMS:
            b = t % _TC_NB
            if t >= _TC_NB:
                pltpu.make_async_copy(bufs[b], dstref(t - _TC_NB),
                                      osems[b]).wait()
            pltpu.async_copy(src(t), bufs[b], isems[b])
        if 2 <= t < _TC_NITEMS + 2:
            tb = (t - 2) % _TC_NB
            pltpu.make_async_copy(src(t - 2), bufs[tb], isems[tb]).wait()
            pltpu.async_copy(bufs[tb], dstref(t - 2), osems[tb])
    for t in range(_TC_NITEMS - _TC_NB, _TC_NITEMS):
        b = t % _TC_NB
        pltpu.make_async_copy(bufs[b], dstref(t), osems[b]).wait()


def _tc_fetch(w13, eids, sids):
    return pl.pallas_call(
        _tc_body,
        grid_spec=pltpu.PrefetchScalarGridSpec(
            num_scalar_prefetch=2,
            grid=(),
            in_specs=[pl.BlockSpec(memory_space=pltpu.MemorySpace.HBM)],
            out_specs=pl.BlockSpec(memory_space=pltpu.MemorySpace.HBM),
            scratch_shapes=(
                [pltpu.VMEM((_TC_PR, _DM), jnp.float32)] * _TC_NB
                + [pltpu.SemaphoreType.DMA] * (2 * _TC_NB)),
        ),
        out_shape=jax.ShapeDtypeStruct((_NUM_SLOTS, _ROWS13, _DM),
                                       jnp.float32),
    )(eids, sids, w13)


@jax.jit
def _fetch(w13, b13, w2, b2, eids, sids):
    o2, ob13, ob2 = _sc_fetch(w2, b13, b2, eids, sids)
    o13 = _tc_fetch(w13, eids, sids)
    return o13, ob13, o2, ob2


def kernel(w13_weight, w13_bias, w2_weight, w2_bias, expert_ids, slot_ids):
    expert_ids = expert_ids.reshape(-1).astype(jnp.int32)
    slot_ids = slot_ids.reshape(-1).astype(jnp.int32)
    o13, ob13, o2, ob2 = _fetch(w13_weight, w13_bias, w2_weight, w2_bias,
                                expert_ids, slot_ids)
    return (o13, ob13, o2, ob2)
